# baseline (speedup 1.0000x reference)
import functools

import jax
import jax.numpy as jnp
from jax import lax
from jax.experimental import pallas as pl
from jax.experimental.pallas import tpu as pltpu
from jax.experimental.pallas import tpu_sc as plsc

N_NODES = 10000
N_HEDGES = 5000
N_INC = 320000
D = 128

NC = 2
NS = 16
NW = NC * NS

PER_W = N_INC // NW
J = 100
B = 100
H = 2
JH = J // H
NBUF = 2
assert J * B == PER_W and JH % NBUF == 0

P_HEDGE = 5120
P_NODE = 10112


def _make_segsum(n_table, n_dst_pad):
  mesh = plsc.VectorSubcoreMesh(core_axis_name="c", subcore_axis_name="s")
  rows_per_tile = n_dst_pad // NS

  @functools.partial(
      pl.kernel,
      out_type=jax.ShapeDtypeStruct((NC, n_dst_pad, D), jnp.float32),
      mesh=mesh,
      scratch_types=[
          pltpu.VMEM((JH, B), jnp.int32),
          pltpu.VMEM((JH, B), jnp.int32),
          [pltpu.VMEM((B, D), jnp.float32)] * NBUF,
          pltpu.VMEM((8, D), jnp.float32),
          pltpu.VMEM_SHARED((n_dst_pad, D), jnp.float32),
          [pltpu.SemaphoreType.DMA] * NBUF,
          [pltpu.SemaphoreType.DMA] * NBUF,
      ],
  )
  def seg(table_hbm, sidx_hbm, didx_hbm, out_hbm,
          sidx_l, didx_l, bufs, zbuf, accum, gsem, ssem):
    cid = lax.axis_index("c")
    sid = lax.axis_index("s")
    wid = cid * NS + sid

    d_si = pltpu.async_copy(sidx_hbm.at[wid, 0], sidx_l, gsem[1])
    d_di = pltpu.async_copy(didx_hbm.at[wid, 0], didx_l, gsem[1])
    z16 = jnp.zeros((16,), jnp.float32)
    for r in range(8):
      for c in range(D // 16):
        zbuf[r, pl.ds(c * 16, 16)] = z16
    n_z = rows_per_tile // 8
    for z0 in range(0, n_z, 8):
      zdescs = []
      for zi in range(z0, min(z0 + 8, n_z)):
        zdescs.append(pltpu.async_copy(
            zbuf, accum.at[pl.ds(sid * rows_per_tile + zi * 8, 8)], gsem[0]))
      for zd in zdescs:
        zd.wait()
    d_si.wait()
    d_di.wait()
    pltpu.async_copy(table_hbm.at[sidx_l.at[0]], bufs[0], gsem[0])
    pltpu.async_copy(table_hbm.at[sidx_l.at[1]], bufs[1], gsem[1])
    plsc.subcore_barrier()

    for h in range(H):
      if h > 0:
        pltpu.sync_copy(sidx_hbm.at[wid, h], sidx_l)
        pltpu.sync_copy(didx_hbm.at[wid, h], didx_l)
        pltpu.async_copy(table_hbm.at[sidx_l.at[0]], bufs[0], gsem[0])
        pltpu.async_copy(table_hbm.at[sidx_l.at[1]], bufs[1], gsem[1])

      def chunk(j, carry):
        c0 = 2 * j
        pltpu.make_async_copy(table_hbm.at[sidx_l.at[c0]], bufs[0],
                              gsem[0]).wait()
        s0 = pltpu.async_copy(bufs[0], accum.at[didx_l.at[c0]], ssem[0],
                              add=True)
        pltpu.make_async_copy(table_hbm.at[sidx_l.at[c0 + 1]], bufs[1],
                              gsem[1]).wait()
        s1 = pltpu.async_copy(bufs[1], accum.at[didx_l.at[c0 + 1]], ssem[1],
                              add=True)
        s0.wait()

        @pl.when(c0 + 2 < JH)
        def _():
          pltpu.async_copy(table_hbm.at[sidx_l.at[c0 + 2]], bufs[0], gsem[0])

        s1.wait()

        @pl.when(c0 + 3 < JH)
        def _():
          pltpu.async_copy(table_hbm.at[sidx_l.at[c0 + 3]], bufs[1], gsem[1])

        return carry

      lax.fori_loop(0, JH // 2, chunk, 0)
    plsc.subcore_barrier()

    pltpu.sync_copy(
        accum.at[pl.ds(sid * rows_per_tile, rows_per_tile)],
        out_hbm.at[cid, pl.ds(sid * rows_per_tile, rows_per_tile)])

  return seg


def _tc_update(n_rows):
  def body(feat_ref, p0_ref, p1_ref, w_ref, b_ref, out_ref):
    msg = p0_ref[pl.ds(0, n_rows), :] + p1_ref[pl.ds(0, n_rows), :]
    w = w_ref[...]
    acc = jnp.dot(feat_ref[...], w[:D], preferred_element_type=jnp.float32)
    acc += jnp.dot(msg, w[D:], preferred_element_type=jnp.float32)
    out_ref[...] = jnp.tanh(acc + b_ref[...])

  return pl.pallas_call(
      body,
      out_shape=jax.ShapeDtypeStruct((n_rows, D), jnp.float32),
  )


def _tc_update_energy(n_rows, want_feat):
  def body(feat_ref, p0_ref, p1_ref, w_ref, b_ref, wm_ref, bm_ref, *outs):
    msg = p0_ref[pl.ds(0, n_rows), :] + p1_ref[pl.ds(0, n_rows), :]
    w = w_ref[...]
    acc = jnp.dot(feat_ref[...], w[:D], preferred_element_type=jnp.float32)
    acc += jnp.dot(msg, w[D:], preferred_element_type=jnp.float32)
    feat = jnp.tanh(acc + b_ref[...])
    e = jnp.tanh(jnp.dot(feat, wm_ref[...], preferred_element_type=jnp.float32)
                 + bm_ref[...])
    if want_feat:
      outs[0][...] = feat
      outs[1][...] = jnp.sum(e).reshape(1, 1)
    else:
      outs[0][...] = jnp.sum(e).reshape(1, 1)

  shapes = ([jax.ShapeDtypeStruct((n_rows, D), jnp.float32)] if want_feat
            else [])
  shapes.append(jax.ShapeDtypeStruct((1, 1), jnp.float32))
  return pl.pallas_call(body, out_shape=shapes)


_seg_to_hedge = _make_segsum(N_NODES, P_HEDGE)
_seg_to_node = _make_segsum(N_HEDGES, P_NODE)
_upd_hedge = _tc_update(N_HEDGES)
_upd_node = _tc_update(N_NODES)
_upd_hedge_e = _tc_update_energy(N_HEDGES, want_feat=True)
_upd_node_e = _tc_update_energy(N_NODES, want_feat=False)


def kernel(node_features, hedge_features, node_index, hedge_index,
           Wh0, bh0, Wn0, bn0, Wh1, bh1, Wn1, bn1, Wnm, bnm, Whm, bhm):
  ni = node_index.reshape(NW, H, JH, B)
  hi = hedge_index.reshape(NW, H, JH, B)
  bh0r = bh0.reshape(1, D)
  bn0r = bn0.reshape(1, D)
  bh1r = bh1.reshape(1, D)
  bn1r = bn1.reshape(1, D)
  bnmr = bnm.reshape(1, 1)
  bhmr = bhm.reshape(1, 1)

  nf, hf = node_features, hedge_features

  m2h = _seg_to_hedge(nf, ni, hi)
  hf = _upd_hedge(hf, m2h[0], m2h[1], Wh0, bh0r)
  m2n = _seg_to_node(hf, hi, ni)
  nf = _upd_node(nf, m2n[0], m2n[1], Wn0, bn0r)

  m2h = _seg_to_hedge(nf, ni, hi)
  hf, hedge_e = _upd_hedge_e(hf, m2h[0], m2h[1], Wh1, bh1r, Whm, bhmr)
  m2n = _seg_to_node(hf, hi, ni)
  (node_e,) = _upd_node_e(nf, m2n[0], m2n[1], Wn1, bn1r, Wnm, bnmr)

  return (node_e + hedge_e).reshape(1)


# speedup vs baseline: 1.2599x; 1.2599x over previous
import functools

import jax
import jax.numpy as jnp
from jax import lax
from jax.experimental import pallas as pl
from jax.experimental.pallas import tpu as pltpu
from jax.experimental.pallas import tpu_sc as plsc

N_NODES = 10000
N_HEDGES = 5000
N_INC = 320000
D = 128

NC = 2
NS = 16
NW = NC * NS

PER_W = N_INC // NW
J = 80
B = 125
H = 4
JH = J // H
NBUF = 2
assert J * B == PER_W and JH % NBUF == 0

P_HEDGE = 5120
P_NODE = 10112


def _make_segsum(n_table, n_dst_pad):
  mesh = plsc.VectorSubcoreMesh(core_axis_name="c", subcore_axis_name="s")
  rows_per_tile = n_dst_pad // NS

  @functools.partial(
      pl.kernel,
      out_type=jax.ShapeDtypeStruct((NC, n_dst_pad, D), jnp.float32),
      mesh=mesh,
      scratch_types=[
          pltpu.VMEM((JH, B), jnp.int32),
          pltpu.VMEM((JH, B), jnp.int32),
          [pltpu.VMEM((B, D), jnp.float32)] * NBUF,
          pltpu.VMEM((8, D), jnp.float32),
          pltpu.VMEM_SHARED((n_dst_pad, D), jnp.float32),
          [pltpu.SemaphoreType.DMA] * NBUF,
      ],
  )
  def seg(table_hbm, sidx_hbm, didx_hbm, out_hbm,
          sidx_l, didx_l, bufs, zbuf, accum, gsem):
    cid = lax.axis_index("c")
    sid = lax.axis_index("s")
    wid = cid * NS + sid

    d_si = pltpu.async_copy(sidx_hbm.at[wid, 0], sidx_l, gsem[1])
    d_di = pltpu.async_copy(didx_hbm.at[wid, 0], didx_l, gsem[1])
    z16 = jnp.zeros((16,), jnp.float32)
    for r in range(8):
      for c in range(D // 16):
        zbuf[r, pl.ds(c * 16, 16)] = z16
    n_z = rows_per_tile // 8
    for z0 in range(0, n_z, 8):
      zdescs = []
      for zi in range(z0, min(z0 + 8, n_z)):
        zdescs.append(pltpu.async_copy(
            zbuf, accum.at[pl.ds(sid * rows_per_tile + zi * 8, 8)], gsem[0]))
      for zd in zdescs:
        zd.wait()
    d_si.wait()
    d_di.wait()
    pltpu.async_copy(table_hbm.at[sidx_l.at[0]], bufs[0], gsem[0])
    plsc.subcore_barrier()

    for h in range(H):
      if h > 0:
        pltpu.sync_copy(sidx_hbm.at[wid, h], sidx_l)
        pltpu.sync_copy(didx_hbm.at[wid, h], didx_l)
        pltpu.async_copy(table_hbm.at[sidx_l.at[0]], bufs[0], gsem[0])

      def chunk(j, carry):
        c0 = 2 * j
        pltpu.async_copy(table_hbm.at[sidx_l.at[c0 + 1]], bufs[1], gsem[1])
        pltpu.make_async_copy(table_hbm.at[sidx_l.at[c0]], bufs[0],
                              gsem[0]).wait()
        pltpu.sync_copy(bufs[0], accum.at[didx_l.at[c0]], add=True)

        @pl.when(c0 + 2 < JH)
        def _():
          pltpu.async_copy(table_hbm.at[sidx_l.at[c0 + 2]], bufs[0], gsem[0])

        pltpu.make_async_copy(table_hbm.at[sidx_l.at[c0 + 1]], bufs[1],
                              gsem[1]).wait()
        pltpu.sync_copy(bufs[1], accum.at[didx_l.at[c0 + 1]], add=True)
        return carry

      lax.fori_loop(0, JH // 2, chunk, 0)
    plsc.subcore_barrier()

    pltpu.sync_copy(
        accum.at[pl.ds(sid * rows_per_tile, rows_per_tile)],
        out_hbm.at[cid, pl.ds(sid * rows_per_tile, rows_per_tile)])

  return seg


def _tc_update(n_rows):
  def body(feat_ref, p0_ref, p1_ref, w_ref, b_ref, out_ref):
    msg = p0_ref[pl.ds(0, n_rows), :] + p1_ref[pl.ds(0, n_rows), :]
    w = w_ref[...]
    acc = jnp.dot(feat_ref[...], w[:D], preferred_element_type=jnp.float32)
    acc += jnp.dot(msg, w[D:], preferred_element_type=jnp.float32)
    out_ref[...] = jnp.tanh(acc + b_ref[...])

  return pl.pallas_call(
      body,
      out_shape=jax.ShapeDtypeStruct((n_rows, D), jnp.float32),
  )


def _tc_update_energy(n_rows, want_feat):
  def body(feat_ref, p0_ref, p1_ref, w_ref, b_ref, wm_ref, bm_ref, *outs):
    msg = p0_ref[pl.ds(0, n_rows), :] + p1_ref[pl.ds(0, n_rows), :]
    w = w_ref[...]
    acc = jnp.dot(feat_ref[...], w[:D], preferred_element_type=jnp.float32)
    acc += jnp.dot(msg, w[D:], preferred_element_type=jnp.float32)
    feat = jnp.tanh(acc + b_ref[...])
    e = jnp.tanh(jnp.dot(feat, wm_ref[...], preferred_element_type=jnp.float32)
                 + bm_ref[...])
    if want_feat:
      outs[0][...] = feat
      outs[1][...] = jnp.sum(e).reshape(1, 1)
    else:
      outs[0][...] = jnp.sum(e).reshape(1, 1)

  shapes = ([jax.ShapeDtypeStruct((n_rows, D), jnp.float32)] if want_feat
            else [])
  shapes.append(jax.ShapeDtypeStruct((1, 1), jnp.float32))
  return pl.pallas_call(body, out_shape=shapes)


_seg_to_hedge = _make_segsum(N_NODES, P_HEDGE)
_seg_to_node = _make_segsum(N_HEDGES, P_NODE)
_upd_hedge = _tc_update(N_HEDGES)
_upd_node = _tc_update(N_NODES)
_upd_hedge_e = _tc_update_energy(N_HEDGES, want_feat=True)
_upd_node_e = _tc_update_energy(N_NODES, want_feat=False)


def kernel(node_features, hedge_features, node_index, hedge_index,
           Wh0, bh0, Wn0, bn0, Wh1, bh1, Wn1, bn1, Wnm, bnm, Whm, bhm):
  ni = node_index.reshape(NW, H, JH, B)
  hi = hedge_index.reshape(NW, H, JH, B)
  bh0r = bh0.reshape(1, D)
  bn0r = bn0.reshape(1, D)
  bh1r = bh1.reshape(1, D)
  bn1r = bn1.reshape(1, D)
  bnmr = bnm.reshape(1, 1)
  bhmr = bhm.reshape(1, 1)

  nf, hf = node_features, hedge_features

  m2h = _seg_to_hedge(nf, ni, hi)
  hf = _upd_hedge(hf, m2h[0], m2h[1], Wh0, bh0r)
  m2n = _seg_to_node(hf, hi, ni)
  nf = _upd_node(nf, m2n[0], m2n[1], Wn0, bn0r)

  m2h = _seg_to_hedge(nf, ni, hi)
  hf, hedge_e = _upd_hedge_e(hf, m2h[0], m2h[1], Wh1, bh1r, Whm, bhmr)
  m2n = _seg_to_node(hf, hi, ni)
  (node_e,) = _upd_node_e(nf, m2n[0], m2n[1], Wn1, bn1r, Wnm, bnmr)

  return (node_e + hedge_e).reshape(1)


# speedup vs baseline: 1.3033x; 1.0345x over previous
import functools

import jax
import jax.numpy as jnp
from jax import lax
from jax.experimental import pallas as pl
from jax.experimental.pallas import tpu as pltpu
from jax.experimental.pallas import tpu_sc as plsc

N_NODES = 10000
N_HEDGES = 5000
N_INC = 320000
D = 128

NC = 2
NS = 16
NW = NC * NS

PER_W = N_INC // NW
NBUF = 2

P_HEDGE = 5120
P_NODE = 10112


def _make_segsum(n_table, n_dst_pad, J, B, H):
  JH = J // H
  assert J * B == PER_W and JH % NBUF == 0
  mesh = plsc.VectorSubcoreMesh(core_axis_name="c", subcore_axis_name="s")
  rows_per_tile = n_dst_pad // NS

  @functools.partial(
      pl.kernel,
      out_type=jax.ShapeDtypeStruct((NC, n_dst_pad, D), jnp.float32),
      mesh=mesh,
      scratch_types=[
          pltpu.VMEM((JH, B), jnp.int32),
          pltpu.VMEM((JH, B), jnp.int32),
          [pltpu.VMEM((B, D), jnp.float32)] * NBUF,
          pltpu.VMEM((8, D), jnp.float32),
          pltpu.VMEM_SHARED((n_dst_pad, D), jnp.float32),
          [pltpu.SemaphoreType.DMA] * NBUF,
      ],
  )
  def seg(table_hbm, sidx_hbm, didx_hbm, out_hbm,
          sidx_l, didx_l, bufs, zbuf, accum, gsem):
    cid = lax.axis_index("c")
    sid = lax.axis_index("s")
    wid = cid * NS + sid

    d_si = pltpu.async_copy(sidx_hbm.at[wid, 0], sidx_l, gsem[1])
    d_di = pltpu.async_copy(didx_hbm.at[wid, 0], didx_l, gsem[1])
    z16 = jnp.zeros((16,), jnp.float32)
    for r in range(8):
      for c in range(D // 16):
        zbuf[r, pl.ds(c * 16, 16)] = z16
    n_z = rows_per_tile // 8
    for z0 in range(0, n_z, 8):
      zdescs = []
      for zi in range(z0, min(z0 + 8, n_z)):
        zdescs.append(pltpu.async_copy(
            zbuf, accum.at[pl.ds(sid * rows_per_tile + zi * 8, 8)], gsem[0]))
      for zd in zdescs:
        zd.wait()
    d_si.wait()
    d_di.wait()
    pltpu.async_copy(table_hbm.at[sidx_l.at[0]], bufs[0], gsem[0])
    plsc.subcore_barrier()

    for h in range(H):
      if h > 0:
        pltpu.sync_copy(sidx_hbm.at[wid, h], sidx_l)
        pltpu.sync_copy(didx_hbm.at[wid, h], didx_l)
        pltpu.async_copy(table_hbm.at[sidx_l.at[0]], bufs[0], gsem[0])

      def chunk(j, carry):
        c0 = 2 * j
        pltpu.async_copy(table_hbm.at[sidx_l.at[c0 + 1]], bufs[1], gsem[1])
        pltpu.make_async_copy(table_hbm.at[sidx_l.at[c0]], bufs[0],
                              gsem[0]).wait()
        pltpu.sync_copy(bufs[0], accum.at[didx_l.at[c0]], add=True)

        @pl.when(c0 + 2 < JH)
        def _():
          pltpu.async_copy(table_hbm.at[sidx_l.at[c0 + 2]], bufs[0], gsem[0])

        pltpu.make_async_copy(table_hbm.at[sidx_l.at[c0 + 1]], bufs[1],
                              gsem[1]).wait()
        pltpu.sync_copy(bufs[1], accum.at[didx_l.at[c0 + 1]], add=True)
        return carry

      lax.fori_loop(0, JH // 2, chunk, 0)
    plsc.subcore_barrier()

    pltpu.sync_copy(
        accum.at[pl.ds(sid * rows_per_tile, rows_per_tile)],
        out_hbm.at[cid, pl.ds(sid * rows_per_tile, rows_per_tile)])

  return seg


def _tc_update(n_rows):
  def body(feat_ref, p0_ref, p1_ref, w_ref, b_ref, out_ref):
    msg = p0_ref[pl.ds(0, n_rows), :] + p1_ref[pl.ds(0, n_rows), :]
    w = w_ref[...]
    acc = jnp.dot(feat_ref[...], w[:D], preferred_element_type=jnp.float32)
    acc += jnp.dot(msg, w[D:], preferred_element_type=jnp.float32)
    out_ref[...] = jnp.tanh(acc + b_ref[...])

  return pl.pallas_call(
      body,
      out_shape=jax.ShapeDtypeStruct((n_rows, D), jnp.float32),
  )


def _tc_update_energy(n_rows, want_feat):
  def body(feat_ref, p0_ref, p1_ref, w_ref, b_ref, wm_ref, bm_ref, *outs):
    msg = p0_ref[pl.ds(0, n_rows), :] + p1_ref[pl.ds(0, n_rows), :]
    w = w_ref[...]
    acc = jnp.dot(feat_ref[...], w[:D], preferred_element_type=jnp.float32)
    acc += jnp.dot(msg, w[D:], preferred_element_type=jnp.float32)
    feat = jnp.tanh(acc + b_ref[...])
    e = jnp.tanh(jnp.dot(feat, wm_ref[...], preferred_element_type=jnp.float32)
                 + bm_ref[...])
    if want_feat:
      outs[0][...] = feat
      outs[1][...] = jnp.sum(e).reshape(1, 1)
    else:
      outs[0][...] = jnp.sum(e).reshape(1, 1)

  shapes = ([jax.ShapeDtypeStruct((n_rows, D), jnp.float32)] if want_feat
            else [])
  shapes.append(jax.ShapeDtypeStruct((1, 1), jnp.float32))
  return pl.pallas_call(body, out_shape=shapes)


JB_H = (80, 125, 1)
JB_N = (100, 100, 2)
_seg_to_hedge = _make_segsum(N_NODES, P_HEDGE, *JB_H)
_seg_to_node = _make_segsum(N_HEDGES, P_NODE, *JB_N)
_upd_hedge = _tc_update(N_HEDGES)
_upd_node = _tc_update(N_NODES)
_upd_hedge_e = _tc_update_energy(N_HEDGES, want_feat=True)
_upd_node_e = _tc_update_energy(N_NODES, want_feat=False)


def kernel(node_features, hedge_features, node_index, hedge_index,
           Wh0, bh0, Wn0, bn0, Wh1, bh1, Wn1, bn1, Wnm, bnm, Whm, bhm):
  jh_h, b_h, h_h = JB_H
  jh_n, b_n, h_n = JB_N
  ni_h = node_index.reshape(NW, h_h, jh_h // h_h, b_h)
  hi_h = hedge_index.reshape(NW, h_h, jh_h // h_h, b_h)
  ni_n = node_index.reshape(NW, h_n, jh_n // h_n, b_n)
  hi_n = hedge_index.reshape(NW, h_n, jh_n // h_n, b_n)
  bh0r = bh0.reshape(1, D)
  bn0r = bn0.reshape(1, D)
  bh1r = bh1.reshape(1, D)
  bn1r = bn1.reshape(1, D)
  bnmr = bnm.reshape(1, 1)
  bhmr = bhm.reshape(1, 1)

  nf, hf = node_features, hedge_features

  m2h = _seg_to_hedge(nf, ni_h, hi_h)
  hf = _upd_hedge(hf, m2h[0], m2h[1], Wh0, bh0r)
  m2n = _seg_to_node(hf, hi_n, ni_n)
  nf = _upd_node(nf, m2n[0], m2n[1], Wn0, bn0r)

  m2h = _seg_to_hedge(nf, ni_h, hi_h)
  hf, hedge_e = _upd_hedge_e(hf, m2h[0], m2h[1], Wh1, bh1r, Whm, bhmr)
  m2n = _seg_to_node(hf, hi_n, ni_n)
  (node_e,) = _upd_node_e(nf, m2n[0], m2n[1], Wn1, bn1r, Wnm, bnmr)

  return (node_e + hedge_e).reshape(1)


# speedup vs baseline: 1.3377x; 1.0264x over previous
import functools

import jax
import jax.numpy as jnp
from jax import lax
from jax.experimental import pallas as pl
from jax.experimental.pallas import tpu as pltpu
from jax.experimental.pallas import tpu_sc as plsc

N_NODES = 10000
N_HEDGES = 5000
N_INC = 320000
D = 128

NC = 2
NS = 16
NW = NC * NS

PER_W = N_INC // NW
NBUF = 2

P_HEDGE = 5120
P_NODE = 10112


def _make_segsum(n_table, n_dst_pad, J, B, H):
  JH = J // H
  assert J * B == PER_W and JH % NBUF == 0
  mesh = plsc.VectorSubcoreMesh(core_axis_name="c", subcore_axis_name="s")
  rows_per_tile = n_dst_pad // NS

  @functools.partial(
      pl.kernel,
      out_type=jax.ShapeDtypeStruct((NC, n_dst_pad, D), jnp.float32),
      mesh=mesh,
      scratch_types=[
          pltpu.VMEM((JH, B), jnp.int32),
          pltpu.VMEM((JH, B), jnp.int32),
          [pltpu.VMEM((B, D), jnp.float32)] * NBUF,
          pltpu.VMEM((8, D), jnp.float32),
          pltpu.VMEM_SHARED((n_dst_pad, D), jnp.float32),
          [pltpu.SemaphoreType.DMA] * NBUF,
      ],
  )
  def seg(table_hbm, sidx_hbm, didx_hbm, out_hbm,
          sidx_l, didx_l, bufs, zbuf, accum, gsem):
    cid = lax.axis_index("c")
    sid = lax.axis_index("s")
    wid = cid * NS + sid

    d_si = pltpu.async_copy(sidx_hbm.at[wid, 0], sidx_l, gsem[1])
    d_di = pltpu.async_copy(didx_hbm.at[wid, 0], didx_l, gsem[1])
    z16 = jnp.zeros((16,), jnp.float32)
    for r in range(8):
      for c in range(D // 16):
        zbuf[r, pl.ds(c * 16, 16)] = z16
    n_z = rows_per_tile // 8
    for z0 in range(0, n_z, 8):
      zdescs = []
      for zi in range(z0, min(z0 + 8, n_z)):
        zdescs.append(pltpu.async_copy(
            zbuf, accum.at[pl.ds(sid * rows_per_tile + zi * 8, 8)], gsem[0]))
      for zd in zdescs:
        zd.wait()
    d_si.wait()
    d_di.wait()
    pltpu.async_copy(table_hbm.at[sidx_l.at[0]], bufs[0], gsem[0])
    plsc.subcore_barrier()

    for h in range(H):
      if h > 0:
        pltpu.sync_copy(sidx_hbm.at[wid, h], sidx_l)
        pltpu.sync_copy(didx_hbm.at[wid, h], didx_l)
        pltpu.async_copy(table_hbm.at[sidx_l.at[0]], bufs[0], gsem[0])

      def chunk(j, carry):
        c0 = 2 * j
        pltpu.async_copy(table_hbm.at[sidx_l.at[c0 + 1]], bufs[1], gsem[1])
        pltpu.make_async_copy(table_hbm.at[sidx_l.at[c0]], bufs[0],
                              gsem[0]).wait()
        pltpu.sync_copy(bufs[0], accum.at[didx_l.at[c0]], add=True)

        @pl.when(c0 + 2 < JH)
        def _():
          pltpu.async_copy(table_hbm.at[sidx_l.at[c0 + 2]], bufs[0], gsem[0])

        pltpu.make_async_copy(table_hbm.at[sidx_l.at[c0 + 1]], bufs[1],
                              gsem[1]).wait()
        pltpu.sync_copy(bufs[1], accum.at[didx_l.at[c0 + 1]], add=True)
        return carry

      lax.fori_loop(0, JH // 2, chunk, 0)
    plsc.subcore_barrier()

    pltpu.sync_copy(
        accum.at[pl.ds(sid * rows_per_tile, rows_per_tile)],
        out_hbm.at[cid, pl.ds(sid * rows_per_tile, rows_per_tile)])

  return seg


def _tc_update(n_rows):
  def body(feat_ref, p0_ref, p1_ref, w_ref, b_ref, out_ref):
    msg = p0_ref[pl.ds(0, n_rows), :] + p1_ref[pl.ds(0, n_rows), :]
    w = w_ref[...]
    acc = jnp.dot(feat_ref[...], w[:D], preferred_element_type=jnp.float32)
    acc += jnp.dot(msg, w[D:], preferred_element_type=jnp.float32)
    out_ref[...] = jnp.tanh(acc + b_ref[...])

  return pl.pallas_call(
      body,
      out_shape=jax.ShapeDtypeStruct((n_rows, D), jnp.float32),
  )


def _tc_update_energy(n_rows, want_feat):
  def body(feat_ref, p0_ref, p1_ref, w_ref, b_ref, wm_ref, bm_ref, *outs):
    msg = p0_ref[pl.ds(0, n_rows), :] + p1_ref[pl.ds(0, n_rows), :]
    w = w_ref[...]
    acc = jnp.dot(feat_ref[...], w[:D], preferred_element_type=jnp.float32)
    acc += jnp.dot(msg, w[D:], preferred_element_type=jnp.float32)
    feat = jnp.tanh(acc + b_ref[...])
    e = jnp.tanh(jnp.dot(feat, wm_ref[...], preferred_element_type=jnp.float32)
                 + bm_ref[...])
    if want_feat:
      outs[0][...] = feat
      outs[1][...] = jnp.sum(e).reshape(1, 1)
    else:
      outs[0][...] = jnp.sum(e).reshape(1, 1)

  shapes = ([jax.ShapeDtypeStruct((n_rows, D), jnp.float32)] if want_feat
            else [])
  shapes.append(jax.ShapeDtypeStruct((1, 1), jnp.float32))
  return pl.pallas_call(body, out_shape=shapes)


JB_H = (80, 125, 1)
JB_N = (80, 125, 2)
_seg_to_hedge = _make_segsum(N_NODES, P_HEDGE, *JB_H)
_seg_to_node = _make_segsum(N_HEDGES, P_NODE, *JB_N)
_upd_hedge = _tc_update(N_HEDGES)
_upd_node = _tc_update(N_NODES)
_upd_hedge_e = _tc_update_energy(N_HEDGES, want_feat=True)
_upd_node_e = _tc_update_energy(N_NODES, want_feat=False)


def kernel(node_features, hedge_features, node_index, hedge_index,
           Wh0, bh0, Wn0, bn0, Wh1, bh1, Wn1, bn1, Wnm, bnm, Whm, bhm):
  jh_h, b_h, h_h = JB_H
  jh_n, b_n, h_n = JB_N
  ni_h = node_index.reshape(NW, h_h, jh_h // h_h, b_h)
  hi_h = hedge_index.reshape(NW, h_h, jh_h // h_h, b_h)
  ni_n = node_index.reshape(NW, h_n, jh_n // h_n, b_n)
  hi_n = hedge_index.reshape(NW, h_n, jh_n // h_n, b_n)
  bh0r = bh0.reshape(1, D)
  bn0r = bn0.reshape(1, D)
  bh1r = bh1.reshape(1, D)
  bn1r = bn1.reshape(1, D)
  bnmr = bnm.reshape(1, 1)
  bhmr = bhm.reshape(1, 1)

  nf, hf = node_features, hedge_features

  m2h = _seg_to_hedge(nf, ni_h, hi_h)
  hf = _upd_hedge(hf, m2h[0], m2h[1], Wh0, bh0r)
  m2n = _seg_to_node(hf, hi_n, ni_n)
  nf = _upd_node(nf, m2n[0], m2n[1], Wn0, bn0r)

  m2h = _seg_to_hedge(nf, ni_h, hi_h)
  hf, hedge_e = _upd_hedge_e(hf, m2h[0], m2h[1], Wh1, bh1r, Whm, bhmr)
  m2n = _seg_to_node(hf, hi_n, ni_n)
  (node_e,) = _upd_node_e(nf, m2n[0], m2n[1], Wn1, bn1r, Wnm, bnmr)

  return (node_e + hedge_e).reshape(1)
